# Initial kernel scaffold; baseline (speedup 1.0000x reference)
import jax
import jax.numpy as jnp
from jax.experimental import pallas as pl


def kernel(x, V, dora_B, dora_A, dora_m, bias):
    raise NotImplementedError("write your pallas kernel here")


# speedup vs baseline: 7.5394x; 7.5394x over previous
import jax
import jax.numpy as jnp
from jax.experimental import pallas as pl
from jax.experimental.pallas import tpu as pltpu

_VMEM_CAP = 60 * 1024 * 1024


def _wprime_kernel(v_ref, b_ref, a_ref, m_ref, w_ref):
    delta_v = jnp.dot(b_ref[...], a_ref[...],
                      preferred_element_type=jnp.float32)
    v_prime = v_ref[...] + delta_v
    sumsq = jnp.sum(v_prime * v_prime, axis=0, keepdims=True)
    scale = m_ref[...] * jax.lax.rsqrt(sumsq)
    w_ref[...] = (v_prime * scale).astype(w_ref.dtype)


def _matmul_bias_kernel(x_ref, w_ref, bias_ref, o_ref):
    xb = x_ref[...].astype(jnp.bfloat16)
    acc = jax.lax.dot_general(
        xb, w_ref[...],
        dimension_numbers=(((1,), (1,)), ((), ())),
        preferred_element_type=jnp.float32)
    o_ref[...] = (acc + bias_ref[...]).astype(o_ref.dtype)


def kernel(x, V, dora_B, dora_A, dora_m, bias):
    *lead, k = x.shape
    d, r = dora_B.shape
    x2 = x.reshape(-1, k)
    m_rows = x2.shape[0]
    if bias is None:
        bias = jnp.zeros((d,), x.dtype)

    tkw = 512 if k % 512 == 0 else k
    w_prime = pl.pallas_call(
        _wprime_kernel,
        out_shape=jax.ShapeDtypeStruct((d, k), jnp.bfloat16),
        grid=(k // tkw,),
        in_specs=[
            pl.BlockSpec((d, tkw), lambda j: (0, j)),
            pl.BlockSpec((d, r),   lambda j: (0, 0)),
            pl.BlockSpec((r, tkw), lambda j: (0, j)),
            pl.BlockSpec((1, tkw), lambda j: (0, j)),
        ],
        out_specs=pl.BlockSpec((d, tkw), lambda j: (0, j)),
        compiler_params=pltpu.CompilerParams(
            dimension_semantics=("parallel",),
            vmem_limit_bytes=_VMEM_CAP,
        ),
    )(V, dora_B, dora_A, dora_m.reshape(1, k))

    tm = 512 if m_rows % 512 == 0 else m_rows
    cost = pl.CostEstimate(
        flops=2 * m_rows * k * d,
        transcendentals=0,
        bytes_accessed=(4 * m_rows * k + 2 * d * k + 4 * m_rows * d + 4 * d),
    )
    out = pl.pallas_call(
        _matmul_bias_kernel,
        out_shape=jax.ShapeDtypeStruct((m_rows, d), x.dtype),
        grid=(m_rows // tm,),
        in_specs=[
            pl.BlockSpec((tm, k), lambda i: (i, 0)),
            pl.BlockSpec((d, k),  lambda i: (0, 0)),
            pl.BlockSpec((1, d),  lambda i: (0, 0)),
        ],
        out_specs=pl.BlockSpec((tm, d), lambda i: (i, 0)),
        compiler_params=pltpu.CompilerParams(
            dimension_semantics=("parallel",),
            vmem_limit_bytes=_VMEM_CAP,
        ),
        cost_estimate=cost,
    )(x2, w_prime, bias.reshape(1, d))

    return out.reshape(*lead, d)


# speedup vs baseline: 8.2848x; 1.0989x over previous
import jax
import jax.numpy as jnp
from jax.experimental import pallas as pl
from jax.experimental.pallas import tpu as pltpu

_VMEM_CAP = 60 * 1024 * 1024


def _dora_kernel(v_ref, b_ref, a_ref, m_ref, x_ref, bias_ref, o_ref, w_ref):
    @pl.when(pl.program_id(0) == 0)
    def _():
        delta_v = jnp.dot(b_ref[...], a_ref[...],
                          preferred_element_type=jnp.float32)
        v_prime = v_ref[...] + delta_v
        sumsq = jnp.sum(v_prime * v_prime, axis=0, keepdims=True)
        scale = m_ref[...] * jax.lax.rsqrt(sumsq)
        w_ref[...] = (v_prime * scale).astype(w_ref.dtype)

    xb = x_ref[...].astype(jnp.bfloat16)
    acc = jax.lax.dot_general(
        xb, w_ref[...],
        dimension_numbers=(((1,), (1,)), ((), ())),
        preferred_element_type=jnp.float32)
    o_ref[...] = (acc + bias_ref[...]).astype(o_ref.dtype)


def kernel(x, V, dora_B, dora_A, dora_m, bias):
    *lead, k = x.shape
    d, r = dora_B.shape
    x2 = x.reshape(-1, k)
    m_rows = x2.shape[0]
    if bias is None:
        bias = jnp.zeros((d,), x.dtype)

    tm = 512 if m_rows % 512 == 0 else m_rows
    cost = pl.CostEstimate(
        flops=2 * m_rows * k * d + 2 * d * r * k,
        transcendentals=k,
        bytes_accessed=(4 * d * k + 4 * m_rows * k + 4 * m_rows * d + 4 * d),
    )
    out = pl.pallas_call(
        _dora_kernel,
        out_shape=jax.ShapeDtypeStruct((m_rows, d), x.dtype),
        grid=(m_rows // tm,),
        in_specs=[
            pl.BlockSpec((d, k),  lambda i: (0, 0)),
            pl.BlockSpec((d, r),  lambda i: (0, 0)),
            pl.BlockSpec((r, k),  lambda i: (0, 0)),
            pl.BlockSpec((1, k),  lambda i: (0, 0)),
            pl.BlockSpec((tm, k), lambda i: (i, 0)),
            pl.BlockSpec((1, d),  lambda i: (0, 0)),
        ],
        out_specs=pl.BlockSpec((tm, d), lambda i: (i, 0)),
        scratch_shapes=[pltpu.VMEM((d, k), jnp.bfloat16)],
        compiler_params=pltpu.CompilerParams(
            dimension_semantics=("arbitrary",),
            vmem_limit_bytes=_VMEM_CAP,
        ),
        cost_estimate=cost,
    )(V, dora_B, dora_A, dora_m.reshape(1, k), x2, bias.reshape(1, d))

    return out.reshape(*lead, d)
